# baseline (speedup 1.0000x reference)
import functools

import jax
import jax.numpy as jnp
from jax import lax
from jax.experimental import pallas as pl
from jax.experimental.pallas import tpu as pltpu
from jax.experimental.pallas import tpu_sc as plsc

NV = 25000
NF = 50000
NE = 150000
H = 512
EDIM = 64

NPF = 53248
BLK = 512
NBLK = NPF // BLK
NE_P = 155648
PASSES = 13
ETILE = NE_P // 16
ETILE_P = ETILE + PASSES * 8
EBLOCKS = ETILE // 512
TROWS = 128
SPAN = 32 * TROWS

_SC_MESH = plsc.VectorSubcoreMesh(core_axis_name="c", subcore_axis_name="s")


@functools.partial(
    pl.kernel,
    out_type=jax.ShapeDtypeStruct((9 * NPF,), jnp.float32),
    mesh=_SC_MESH,
    scratch_types=[
        pltpu.VMEM((4, 128), jnp.int32),
        pltpu.VMEM((4, 128), jnp.float32),
        pltpu.SemaphoreType.DMA,
        pltpu.SemaphoreType.DMA,
    ],
)
def _vertex_gather(vx, vy, vz, fT, out, idxv, outv, semg, semo):
    c = lax.axis_index("c")
    s = lax.axis_index("s")
    base = (s * 2 + c) * (NPF // 32)
    tabs = (vx, vy, vz)
    nch = NPF // 32 // 128
    for k in range(3):
        for ci in range(3):
            row = k * 3 + ci

            def body(j, _, k=k, ci=ci, row=row):
                o = base + j * 512
                hs = []
                for b in range(4):
                    pltpu.sync_copy(
                        fT.at[pl.ds(k * NPF + o + b * 128, 128)], idxv.at[b])
                    hs.append(pltpu.async_copy(
                        tabs[ci].at[idxv.at[b]], outv.at[b], semg))
                ho = []
                for b in range(4):
                    hs[b].wait()
                    ho.append(pltpu.async_copy(
                        outv.at[b],
                        out.at[pl.ds(row * NPF + o + b * 128, 128)], semo))
                for h in ho:
                    h.wait()
                return 0

            lax.fori_loop(0, nch // 4, body, 0)
            for t in range(nch - nch % 4, nch):
                o = base + t * 128
                pltpu.sync_copy(fT.at[pl.ds(k * NPF + o, 128)], idxv.at[0])
                pltpu.async_copy(
                    tabs[ci].at[idxv.at[0]], outv.at[0], semg).wait()
                pltpu.sync_copy(outv.at[0],
                                out.at[pl.ds(row * NPF + o, 128)])


@functools.partial(
    pl.kernel,
    out_type=jax.ShapeDtypeStruct((NPF * 16, EDIM), jnp.float32),
    mesh=_SC_MESH,
    scratch_types=[
        pltpu.VMEM((4, 128), jnp.int32),
        pltpu.VMEM((4, 128, EDIM), jnp.float32),
        pltpu.SemaphoreType.DMA,
        pltpu.SemaphoreType.DMA,
    ],
    compiler_params=pltpu.CompilerParams(
        use_tc_tiling_on_sc=False, needs_layout_passes=False),
)
def _embed_gather(tbl, gflat, out, idxv, rows, semg, semo):
    c = lax.axis_index("c")
    s = lax.axis_index("s")
    w = s * 2 + c
    n_per = NPF * 16 // 32
    base = w * n_per
    roff = (w % 16) * 516

    def group(g, _):
        o = base + g * 512
        hs = []
        for b in range(4):
            pltpu.sync_copy(gflat.at[pl.ds(o + b * 128, 128)], idxv.at[b])
            for v in range(8):
                idxv[b, pl.ds(v * 16, 16)] = (
                    idxv[b, pl.ds(v * 16, 16)] + roff)
            hs.append(pltpu.async_copy(tbl.at[idxv.at[b]], rows.at[b], semg))
        ho = []
        for b in range(4):
            hs[b].wait()
            ho.append(pltpu.async_copy(
                rows.at[b], out.at[pl.ds(o + b * 128, 128)], semo))
        for h in ho:
            h.wait()
        return 0

    lax.fori_loop(0, n_per // 512, group, 0)


@functools.partial(
    pl.kernel,
    out_type=(
        jax.ShapeDtypeStruct((NPF, H), jnp.float32),
        jax.ShapeDtypeStruct((NPF, 16), jnp.float32),
        jax.ShapeDtypeStruct((32 * ETILE_P + 512,), jnp.int32),
        jax.ShapeDtypeStruct((1024,), jnp.int32),
    ),
    mesh=_SC_MESH,
    scratch_types=[
        pltpu.VMEM((TROWS, H + 16), jnp.float32),
        pltpu.VMEM((BLK,), jnp.int32),
        pltpu.VMEM((BLK,), jnp.int32),
        pltpu.VMEM((ETILE_P,), jnp.int32),
        pltpu.VMEM((512,), jnp.int32),
        pltpu.VMEM((544,), jnp.int32),
        pltpu.VMEM((544,), jnp.int32),
        pltpu.VMEM((16, H), jnp.float32),
        pltpu.VMEM((16, H), jnp.float32),
        pltpu.SemaphoreType.DMA,
        pltpu.SemaphoreType.DMA,
    ],
    compiler_params=pltpu.CompilerParams(needs_layout_passes=False),
)
def _edge_agg(qmat, srcv, dstv, agg, cnt, bsrc, bmeta,
              acc, sbuf, dbuf, bufsrc, metabuf, hsrc, hdl,
              grows, growsb, sem, semb):
    c = lax.axis_index("c")
    s = lax.axis_index("s")
    w = s * 2 + c
    iot = lax.iota(jnp.int32, 16)
    zero16 = jnp.zeros((16,), jnp.float32)
    zero16i = jnp.zeros((16,), jnp.int32)
    e0 = jnp.where(iot == 0, 1.0, 0.0)

    ebase = s * ETILE

    def cblk(bk, cntv):
        pltpu.sync_copy(dstv.at[pl.ds(ebase + bk * BLK, BLK)], dbuf)

        def cl(l, cntv):
            bv = dbuf[pl.ds(l * 16, 16)] >> 12
            for b in range(PASSES):
                pc = plsc.all_reduce_population_count(bv == b)
                cntv = cntv + jnp.where(iot == b, pc, 0)
            return cntv

        return lax.fori_loop(0, BLK // 16, cl, cntv)

    cntv = lax.fori_loop(0, EBLOCKS, cblk, zero16i)
    cnt8 = (cntv + 7) & ~7
    startv = plsc.cumsum(cnt8) - cnt8
    metabuf[pl.ds(0, 16)] = startv
    metabuf[pl.ds(16, 16)] = cntv
    pltpu.sync_copy(metabuf.at[pl.ds(0, 32)],
                    bmeta.at[pl.ds(c * 512 + s * 32, 32)])

    def rblk(bk, runv):
        pltpu.sync_copy(srcv.at[pl.ds(ebase + bk * BLK, BLK)], sbuf)
        pltpu.sync_copy(dstv.at[pl.ds(ebase + bk * BLK, BLK)], dbuf)

        def rl(l, runv):
            dv = dbuf[pl.ds(l * 16, 16)]
            sv = sbuf[pl.ds(l * 16, 16)]
            pk = sv | (dv << 16)
            bv = dv >> 12
            for b in range(PASSES):
                m = bv == b
                mi = m.astype(jnp.int32)
                incl = plsc.cumsum(mi)
                pos = (startv[b] + runv[b]) + incl - mi
                plsc.store_scatter(bufsrc, [pos], pk, mask=m)
                runv = runv + jnp.where(iot == b, incl[15], 0)
            return runv

        return lax.fori_loop(0, BLK // 16, rl, runv)

    lax.fori_loop(0, EBLOCKS, rblk, zero16i)
    pltpu.sync_copy(bufsrc, bsrc.at[pl.ds((c * 16 + s) * ETILE_P, ETILE_P)])
    plsc.subcore_barrier()
    pltpu.sync_copy(bmeta.at[pl.ds(c * 512, 512)], metabuf)

    def accumulate(i, dlv, buf):
        dl = dlv[i]

        for j in range(H // 16):
            plsc.addupdate(acc.at[dl, pl.ds(j * 16, 16)],
                           buf[i, pl.ds(j * 16, 16)])
        plsc.addupdate(acc.at[dl, pl.ds(H, 16)], e0)

    def pass_body(p, _):
        mybase = p * SPAN + w * TROWS

        def zp(i, _):
            for j in range((H + 16) // 16):
                acc[i, pl.ds(j * 16, 16)] = zero16
            return 0

        lax.fori_loop(0, TROWS, zp, 0)
        pm = iot == p

        def st_body(st, off):
            mv_s = metabuf[pl.ds(st * 32, 16)]
            mv_c = metabuf[pl.ds(st * 32 + 16, 16)]
            stt = pl.multiple_of(jnp.sum(jnp.where(pm, mv_s, 0)), 8)
            cn = jnp.sum(jnp.where(pm, mv_c, 0))
            segbase = (c * 16 + st) * ETILE_P + stt

            def chunk(kv):
                k, off = kv
                pltpu.sync_copy(bsrc.at[pl.ds(segbase + k * BLK, BLK)], sbuf)

                def lane_body(l, off):
                    pk = sbuf[pl.ds(l * 16, 16)]
                    dv = lax.shift_right_logical(pk, 16)
                    dl = dv - mybase
                    gi = k * BLK + l * 16
                    okm = (dl >= 0) & (dl < TROWS) & ((gi + iot) < cn)

                    def compact():
                        sv = pk & 0xFFFF
                        oki = okm.astype(jnp.int32)
                        incl = plsc.cumsum(oki)
                        pos = off + incl - oki
                        plsc.store_scatter(hsrc, [pos], sv, mask=okm)
                        plsc.store_scatter(hdl, [pos], dl, mask=okm)
                        return off + incl[15]

                    return lax.cond(jnp.any(okm), compact, lambda: off)

                off = lax.fori_loop(0, BLK // 16, lane_body, off,
                                    unroll=2)
                nfull = off // 16

                def drain2(k2):
                    a = k2 * 2
                    b = a + 1
                    ha = pltpu.async_copy(
                        qmat.at[hsrc.at[pl.ds(a * 16, 16)]], grows, sem)

                    @pl.when(b < nfull)
                    def _():
                        pltpu.async_copy(
                            qmat.at[hsrc.at[pl.ds(b * 16, 16)]], growsb, semb)

                    ha.wait()
                    dlv = hdl[pl.ds(a * 16, 16)]
                    for i in range(16):
                        accumulate(i, dlv, grows)

                    @pl.when(b < nfull)
                    def _():
                        pltpu.make_async_copy(
                            qmat.at[pl.ds(0, 16)], growsb, semb).wait()
                        dlvb = hdl[pl.ds(b * 16, 16)]
                        for i in range(16):
                            accumulate(i, dlvb, growsb)

                    return k2 + 1

                lax.while_loop(lambda k2: k2 * 2 < nfull, drain2, 0)
                rem = off - nfull * 16
                lsr = hsrc[pl.ds(nfull * 16, 16)]
                ldl = hdl[pl.ds(nfull * 16, 16)]
                mrem = iot < rem
                plsc.store_scatter(hsrc, [iot], lsr, mask=mrem)
                plsc.store_scatter(hdl, [iot], ldl, mask=mrem)
                return k + 1, rem

            _, off = lax.while_loop(lambda kv: kv[0] * BLK < cn, chunk,
                                    (0, off))
            return off

        off = lax.fori_loop(0, 16, st_body, 0)

        mok = iot < off
        srv = jnp.where(mok, hsrc[pl.ds(0, 16)], 0)
        hsrc[pl.ds(0, 16)] = srv
        pltpu.async_copy(qmat.at[hsrc.at[pl.ds(0, 16)]], grows, sem).wait()
        tdlv = hdl[pl.ds(0, 16)]

        def tail(i):
            jax.lax.switch(i, [lambda i=i: accumulate(i, tdlv, grows)
                               for i in range(16)])
            return i + 1

        lax.while_loop(lambda i: i < off, tail, 0)

        pltpu.sync_copy(acc.at[pl.ds(0, TROWS), pl.ds(0, H)],
                        agg.at[pl.ds(mybase, TROWS)])
        pltpu.sync_copy(acc.at[pl.ds(0, TROWS), pl.ds(H, 16)],
                        cnt.at[pl.ds(mybase, TROWS)])
        return 0

    lax.fori_loop(0, PASSES, pass_body, 0)


def _geom_body(fc_ref, out_ref):
    f = fc_ref[...]
    eps = 1e-8

    def row(i):
        return f[i:i + 1, :]

    v0 = (row(0), row(1), row(2))
    v1 = (row(3), row(4), row(5))
    v2 = (row(6), row(7), row(8))
    fm = row(9)

    def sub3(a, b):
        return tuple(x - y for x, y in zip(a, b))

    def dot3(a, b):
        return (a[0] * b[0] + a[1] * b[1]) + a[2] * b[2]

    def norm3(a):
        return jnp.sqrt(dot3(a, a))

    def ang(a, cc, d):
        e1 = sub3(cc, a)
        e2 = sub3(d, a)
        cosang = dot3(e1, e2) / (norm3(e1) * norm3(e2) + eps)
        x = jnp.clip(cosang, -1.0 + 1e-7, 1.0 - 1e-7)
        return 2.0 * jnp.arctan2(jnp.sqrt(1.0 - x * x), 1.0 + x)

    a0 = ang(v0, v1, v2)
    a1 = ang(v1, v2, v0)
    a2 = ang(v2, v0, v1)
    u = sub3(v1, v0)
    w = sub3(v2, v0)
    cx = u[1] * w[2] - u[2] * w[1]
    cy = u[2] * w[0] - u[0] * w[2]
    cz = u[0] * w[1] - u[1] * w[0]
    cn = jnp.sqrt((cx * cx + cy * cy) + cz * cz)
    area = 0.5 * cn
    nx = cx / (cn + eps)
    ny = cy / (cn + eps)
    nz = cz / (cn + eps)

    def quant(x, high, low):
        t = (x - low) / (high - low)
        return jnp.clip(jnp.floor(t * 128.0), 0.0, 127.0).astype(jnp.int32) + 1

    mask = fm > 0.5
    import numpy as _np
    rows = []
    for i in range(9):
        rows.append(jnp.where(mask, quant(row(i), 1.0, 0.0), 0))
    for a in (a0, a1, a2):
        rows.append(jnp.where(mask, quant(a, _np.pi, 0.0), 0) + 129)
    for nrm in (nx, ny, nz):
        rows.append(jnp.where(mask, quant(nrm, 1.0, -1.0), 0) + 258)
    rows.append(jnp.where(mask, quant(area, 0.5, 0.0), 0) + 387)
    out_ref[...] = jnp.concatenate(rows, axis=0)


def _proj_body(feats_ref, wp_ref, wsp_ref, wl_ref, wr_ref, aux_ref,
               q_ref, r_ref):
    fb = feats_ref[...].astype(jnp.bfloat16)
    x = jnp.dot(fb, wp_ref[...], preferred_element_type=jnp.float32)
    x = x + aux_ref[0:1, :]
    h = jax.nn.gelu(x)
    hb = h.astype(jnp.bfloat16)
    p = jnp.maximum(
        jnp.dot(hb, wsp_ref[...], preferred_element_type=jnp.float32)
        + aux_ref[1:2, :], 0.0)
    q_ref[...] = jnp.dot(p.astype(jnp.bfloat16), wl_ref[...],
                         preferred_element_type=jnp.float32)
    r_ref[...] = jnp.dot(hb, wr_ref[...],
                         preferred_element_type=jnp.float32)


def _final_body(agg_ref, r_ref, cnt_ref, fm_ref, aux_ref, out_ref):
    x = agg_ref[...] / jnp.maximum(cnt_ref[...], 1.0)
    x = x + aux_ref[0:1, :] + r_ref[...]
    nrm = jnp.sqrt(jnp.sum(x * x, axis=1, keepdims=True))
    x = x / jnp.maximum(nrm, 1e-12)
    x = jax.nn.gelu(x)
    mu = jnp.mean(x, axis=1, keepdims=True)
    var = jnp.mean((x - mu) ** 2, axis=1, keepdims=True)
    x = (x - mu) / jnp.sqrt(var + 1e-5) * aux_ref[1:2, :] + aux_ref[2:3, :]
    out_ref[...] = x * fm_ref[...]


def kernel(vertices, faces, edges, face_masks, edge_masks, embed_vertex,
           embed_angle, embed_norm, embed_area, proj_W, proj_b, sage_proj_W,
           sage_proj_b, sage_Wl, sage_bl, sage_Wr, ln_gamma, ln_beta):
    v = vertices[0]
    vx, vy, vz = v[:, 0], v[:, 1], v[:, 2]
    fT = jnp.pad(jnp.transpose(faces[0]).astype(jnp.int32),
                 ((0, 0), (0, NPF - NF))).reshape(-1)
    fc9 = _vertex_gather(vx, vy, vz, fT).reshape(9, NPF)

    fmp = jnp.pad(face_masks[0].astype(jnp.float32), (0, NPF - NF))
    fc10 = jnp.concatenate([fc9, fmp[None, :]], axis=0)

    gidx = pl.pallas_call(
        _geom_body,
        grid=(NBLK,),
        in_specs=[pl.BlockSpec((10, BLK), lambda i: (0, i))],
        out_specs=pl.BlockSpec((16, BLK), lambda i: (0, i)),
        out_shape=jax.ShapeDtypeStruct((16, NPF), jnp.int32),
    )(fc10)

    gflat = jnp.transpose(gidx).reshape(-1)
    tbl = jnp.concatenate(
        [embed_vertex, embed_angle, embed_norm, embed_area], axis=0)
    feats = _embed_gather(jnp.tile(tbl, (16, 1)),
                          gflat).reshape(NPF, 16 * EDIM)

    aux_d = jnp.zeros((8, H), jnp.float32).at[0].set(proj_b).at[1].set(
        sage_proj_b)
    q, r = pl.pallas_call(
        _proj_body,
        grid=(NBLK,),
        in_specs=[
            pl.BlockSpec((BLK, 16 * EDIM), lambda i: (i, 0)),
            pl.BlockSpec((16 * EDIM, H), lambda i: (0, 0)),
            pl.BlockSpec((H, H), lambda i: (0, 0)),
            pl.BlockSpec((H, H), lambda i: (0, 0)),
            pl.BlockSpec((H, H), lambda i: (0, 0)),
            pl.BlockSpec((8, H), lambda i: (0, 0)),
        ],
        out_specs=[pl.BlockSpec((BLK, H), lambda i: (i, 0))] * 2,
        out_shape=[jax.ShapeDtypeStruct((NPF, H), jnp.float32)] * 2,
    )(feats, proj_W.astype(jnp.bfloat16), sage_proj_W.astype(jnp.bfloat16),
      sage_Wl.astype(jnp.bfloat16), sage_Wr.astype(jnp.bfloat16), aux_d)

    e = edges[0].astype(jnp.int32)
    src_p = jnp.pad(e[:, 0], (0, NE_P - NE))
    dst_eff = jnp.where(edge_masks[0], e[:, 1], 61440)
    dst_p = jnp.pad(dst_eff, (0, NE_P - NE), constant_values=61440)
    aggv, cnt2, _, _ = _edge_agg(q, src_p, dst_p)
    cntv = cnt2[:, 0]

    aux_f = jnp.zeros((8, H), jnp.float32).at[0].set(sage_bl).at[1].set(
        ln_gamma).at[2].set(ln_beta)
    out = pl.pallas_call(
        _final_body,
        grid=(NBLK,),
        in_specs=[
            pl.BlockSpec((BLK, H), lambda i: (i, 0)),
            pl.BlockSpec((BLK, H), lambda i: (i, 0)),
            pl.BlockSpec((BLK, 1), lambda i: (i, 0)),
            pl.BlockSpec((BLK, 1), lambda i: (i, 0)),
            pl.BlockSpec((8, H), lambda i: (0, 0)),
        ],
        out_specs=pl.BlockSpec((BLK, H), lambda i: (i, 0)),
        out_shape=jax.ShapeDtypeStruct((NPF, H), jnp.float32),
    )(aggv, r, cntv.reshape(NPF, 1), fmp.reshape(NPF, 1), aux_f)

    return out[:NF].reshape(1, NF, H)


# speedup vs baseline: 1.0054x; 1.0054x over previous
import functools

import jax
import jax.numpy as jnp
from jax import lax
from jax.experimental import pallas as pl
from jax.experimental.pallas import tpu as pltpu
from jax.experimental.pallas import tpu_sc as plsc

NV = 25000
NF = 50000
NE = 150000
H = 512
EDIM = 64

NPF = 53248
BLK = 512
NBLK = NPF // BLK
NE_P = 155648
PASSES = 13
ETILE = NE_P // 16
ETILE_P = ETILE + PASSES * 8
EBLOCKS = ETILE // 512
TROWS = 128
SPAN = 32 * TROWS

_SC_MESH = plsc.VectorSubcoreMesh(core_axis_name="c", subcore_axis_name="s")


@functools.partial(
    pl.kernel,
    out_type=jax.ShapeDtypeStruct((9 * NPF,), jnp.float32),
    mesh=_SC_MESH,
    scratch_types=[
        pltpu.VMEM((4, 128), jnp.int32),
        pltpu.VMEM((4, 128), jnp.float32),
        pltpu.SemaphoreType.DMA,
        pltpu.SemaphoreType.DMA,
    ],
)
def _vertex_gather(vx, vy, vz, fT, out, idxv, outv, semg, semo):
    c = lax.axis_index("c")
    s = lax.axis_index("s")
    base = (s * 2 + c) * (NPF // 32)
    tabs = (vx, vy, vz)
    nch = NPF // 32 // 128
    for k in range(3):
        for ci in range(3):
            row = k * 3 + ci

            def body(j, _, k=k, ci=ci, row=row):
                o = base + j * 512
                hs = []
                for b in range(4):
                    pltpu.sync_copy(
                        fT.at[pl.ds(k * NPF + o + b * 128, 128)], idxv.at[b])
                    hs.append(pltpu.async_copy(
                        tabs[ci].at[idxv.at[b]], outv.at[b], semg))
                ho = []
                for b in range(4):
                    hs[b].wait()
                    ho.append(pltpu.async_copy(
                        outv.at[b],
                        out.at[pl.ds(row * NPF + o + b * 128, 128)], semo))
                for h in ho:
                    h.wait()
                return 0

            lax.fori_loop(0, nch // 4, body, 0)
            for t in range(nch - nch % 4, nch):
                o = base + t * 128
                pltpu.sync_copy(fT.at[pl.ds(k * NPF + o, 128)], idxv.at[0])
                pltpu.async_copy(
                    tabs[ci].at[idxv.at[0]], outv.at[0], semg).wait()
                pltpu.sync_copy(outv.at[0],
                                out.at[pl.ds(row * NPF + o, 128)])


@functools.partial(
    pl.kernel,
    out_type=jax.ShapeDtypeStruct((NPF * 16, EDIM), jnp.float32),
    mesh=_SC_MESH,
    scratch_types=[
        pltpu.VMEM((4, 128), jnp.int32),
        pltpu.VMEM((4, 128, EDIM), jnp.float32),
        pltpu.SemaphoreType.DMA,
        pltpu.SemaphoreType.DMA,
    ],
    compiler_params=pltpu.CompilerParams(
        use_tc_tiling_on_sc=False, needs_layout_passes=False),
)
def _embed_gather(tbl, gflat, out, idxv, rows, semg, semo):
    c = lax.axis_index("c")
    s = lax.axis_index("s")
    w = s * 2 + c
    n_per = NPF * 16 // 32
    base = w * n_per
    roff = (w % 16) * 516

    def group(g, _):
        o = base + g * 512
        hs = []
        for b in range(4):
            pltpu.sync_copy(gflat.at[pl.ds(o + b * 128, 128)], idxv.at[b])
            for v in range(8):
                idxv[b, pl.ds(v * 16, 16)] = (
                    idxv[b, pl.ds(v * 16, 16)] + roff)
            hs.append(pltpu.async_copy(tbl.at[idxv.at[b]], rows.at[b], semg))
        ho = []
        for b in range(4):
            hs[b].wait()
            ho.append(pltpu.async_copy(
                rows.at[b], out.at[pl.ds(o + b * 128, 128)], semo))
        for h in ho:
            h.wait()
        return 0

    lax.fori_loop(0, n_per // 512, group, 0)


@functools.partial(
    pl.kernel,
    out_type=(
        jax.ShapeDtypeStruct((NPF, H), jnp.float32),
        jax.ShapeDtypeStruct((NPF, 16), jnp.float32),
        jax.ShapeDtypeStruct((32 * ETILE_P + 512,), jnp.int32),
        jax.ShapeDtypeStruct((1024,), jnp.int32),
    ),
    mesh=_SC_MESH,
    scratch_types=[
        pltpu.VMEM((TROWS, H + 16), jnp.float32),
        pltpu.VMEM((BLK,), jnp.int32),
        pltpu.VMEM((BLK,), jnp.int32),
        pltpu.VMEM((ETILE_P,), jnp.int32),
        pltpu.VMEM((512,), jnp.int32),
        pltpu.VMEM((544,), jnp.int32),
        pltpu.VMEM((544,), jnp.int32),
        pltpu.VMEM((16, H), jnp.float32),
        pltpu.VMEM((16, H), jnp.float32),
        pltpu.SemaphoreType.DMA,
        pltpu.SemaphoreType.DMA,
    ],
    compiler_params=pltpu.CompilerParams(needs_layout_passes=False),
)
def _edge_agg(qmat, srcv, dstv, agg, cnt, bsrc, bmeta,
              acc, sbuf, dbuf, bufsrc, metabuf, hsrc, hdl,
              grows, growsb, sem, semb):
    c = lax.axis_index("c")
    s = lax.axis_index("s")
    w = s * 2 + c
    iot = lax.iota(jnp.int32, 16)
    zero16 = jnp.zeros((16,), jnp.float32)
    zero16i = jnp.zeros((16,), jnp.int32)
    e0 = jnp.where(iot == 0, 1.0, 0.0)

    ebase = s * ETILE

    def cblk(bk, cntv):
        pltpu.sync_copy(dstv.at[pl.ds(ebase + bk * BLK, BLK)], dbuf)

        def cl(l, cntv):
            bv = dbuf[pl.ds(l * 16, 16)] >> 12
            for b in range(PASSES):
                pc = plsc.all_reduce_population_count(bv == b)
                cntv = cntv + jnp.where(iot == b, pc, 0)
            return cntv

        return lax.fori_loop(0, BLK // 16, cl, cntv)

    cntv = lax.fori_loop(0, EBLOCKS, cblk, zero16i)
    cnt8 = (cntv + 7) & ~7
    startv = plsc.cumsum(cnt8) - cnt8
    metabuf[pl.ds(0, 16)] = startv
    metabuf[pl.ds(16, 16)] = cntv
    pltpu.sync_copy(metabuf.at[pl.ds(0, 32)],
                    bmeta.at[pl.ds(c * 512 + s * 32, 32)])

    def rblk(bk, runv):
        pltpu.sync_copy(srcv.at[pl.ds(ebase + bk * BLK, BLK)], sbuf)
        pltpu.sync_copy(dstv.at[pl.ds(ebase + bk * BLK, BLK)], dbuf)

        def rl(l, runv):
            dv = dbuf[pl.ds(l * 16, 16)]
            sv = sbuf[pl.ds(l * 16, 16)]
            pk = sv | (dv << 16)
            bv = dv >> 12
            for b in range(PASSES):
                m = bv == b
                mi = m.astype(jnp.int32)
                incl = plsc.cumsum(mi)
                pos = (startv[b] + runv[b]) + incl - mi
                plsc.store_scatter(bufsrc, [pos], pk, mask=m)
                runv = runv + jnp.where(iot == b, incl[15], 0)
            return runv

        return lax.fori_loop(0, BLK // 16, rl, runv)

    lax.fori_loop(0, EBLOCKS, rblk, zero16i)
    pltpu.sync_copy(bufsrc, bsrc.at[pl.ds((c * 16 + s) * ETILE_P, ETILE_P)])
    plsc.subcore_barrier()
    pltpu.sync_copy(bmeta.at[pl.ds(c * 512, 512)], metabuf)

    def accumulate(i, dlv, buf):
        dl = dlv[i]

        for j in range(H // 16):
            plsc.addupdate(acc.at[dl, pl.ds(j * 16, 16)],
                           buf[i, pl.ds(j * 16, 16)])
        plsc.addupdate(acc.at[dl, pl.ds(H, 16)], e0)

    def pass_body(p, _):
        mybase = p * SPAN + w * TROWS

        def zp(i, _):
            for j in range((H + 16) // 16):
                acc[i, pl.ds(j * 16, 16)] = zero16
            return 0

        lax.fori_loop(0, TROWS, zp, 0)
        pm = iot == p

        def st_body(st, off):
            mv_s = metabuf[pl.ds(st * 32, 16)]
            mv_c = metabuf[pl.ds(st * 32 + 16, 16)]
            stt = pl.multiple_of(jnp.sum(jnp.where(pm, mv_s, 0)), 8)
            cn = jnp.sum(jnp.where(pm, mv_c, 0))
            segbase = (c * 16 + st) * ETILE_P + stt

            def chunk(kv):
                k, off = kv
                pltpu.sync_copy(bsrc.at[pl.ds(segbase + k * BLK, BLK)], sbuf)

                def lane_body(l, off):
                    pk = sbuf[pl.ds(l * 16, 16)]
                    dv = lax.shift_right_logical(pk, 16)
                    dl = dv - mybase
                    gi = k * BLK + l * 16
                    okm = (dl >= 0) & (dl < TROWS) & ((gi + iot) < cn)

                    def compact():
                        sv = pk & 0xFFFF
                        oki = okm.astype(jnp.int32)
                        incl = plsc.cumsum(oki)
                        pos = off + incl - oki
                        plsc.store_scatter(hsrc, [pos], sv, mask=okm)
                        plsc.store_scatter(hdl, [pos], dl, mask=okm)
                        return off + incl[15]

                    return lax.cond(jnp.any(okm), compact, lambda: off)

                off = lax.fori_loop(0, BLK // 16, lane_body, off,
                                    unroll=2)
                nfull = off // 16

                def drain2(k2):
                    a = k2 * 2
                    b = a + 1
                    ha = pltpu.async_copy(
                        qmat.at[hsrc.at[pl.ds(a * 16, 16)]], grows, sem)

                    @pl.when(b < nfull)
                    def _():
                        pltpu.async_copy(
                            qmat.at[hsrc.at[pl.ds(b * 16, 16)]], growsb, semb)

                    ha.wait()
                    dlv = hdl[pl.ds(a * 16, 16)]
                    for i in range(16):
                        accumulate(i, dlv, grows)

                    @pl.when(b < nfull)
                    def _():
                        pltpu.make_async_copy(
                            qmat.at[pl.ds(0, 16)], growsb, semb).wait()
                        dlvb = hdl[pl.ds(b * 16, 16)]
                        for i in range(16):
                            accumulate(i, dlvb, growsb)

                    return k2 + 1

                lax.while_loop(lambda k2: k2 * 2 < nfull, drain2, 0)
                rem = off - nfull * 16
                lsr = hsrc[pl.ds(nfull * 16, 16)]
                ldl = hdl[pl.ds(nfull * 16, 16)]
                mrem = iot < rem
                plsc.store_scatter(hsrc, [iot], lsr, mask=mrem)
                plsc.store_scatter(hdl, [iot], ldl, mask=mrem)
                return k + 1, rem

            _, off = lax.while_loop(lambda kv: kv[0] * BLK < cn, chunk,
                                    (0, off))
            return off

        off = lax.fori_loop(0, 16, st_body, 0)

        mok = iot < off
        srv = jnp.where(mok, hsrc[pl.ds(0, 16)], 0)
        hsrc[pl.ds(0, 16)] = srv
        pltpu.async_copy(qmat.at[hsrc.at[pl.ds(0, 16)]], grows, sem).wait()
        tdlv = hdl[pl.ds(0, 16)]

        def tail(i):
            jax.lax.switch(i, [lambda i=i: accumulate(i, tdlv, grows)
                               for i in range(16)])
            return i + 1

        lax.while_loop(lambda i: i < off, tail, 0)

        pltpu.sync_copy(acc.at[pl.ds(0, TROWS), pl.ds(0, H)],
                        agg.at[pl.ds(mybase, TROWS)])
        pltpu.sync_copy(acc.at[pl.ds(0, TROWS), pl.ds(H, 16)],
                        cnt.at[pl.ds(mybase, TROWS)])
        return 0

    lax.fori_loop(0, PASSES, pass_body, 0)


def _geom_body(fc_ref, out_ref):
    f = fc_ref[...]
    eps = 1e-8

    def row(i):
        return f[i:i + 1, :]

    v0 = (row(0), row(1), row(2))
    v1 = (row(3), row(4), row(5))
    v2 = (row(6), row(7), row(8))
    fm = row(9)

    def sub3(a, b):
        return tuple(x - y for x, y in zip(a, b))

    def dot3(a, b):
        return (a[0] * b[0] + a[1] * b[1]) + a[2] * b[2]

    def norm3(a):
        return jnp.sqrt(dot3(a, a))

    def ang(a, cc, d):
        e1 = sub3(cc, a)
        e2 = sub3(d, a)
        cosang = dot3(e1, e2) / (norm3(e1) * norm3(e2) + eps)
        x = jnp.clip(cosang, -1.0 + 1e-7, 1.0 - 1e-7)
        return 2.0 * jnp.arctan2(jnp.sqrt(1.0 - x * x), 1.0 + x)

    a0 = ang(v0, v1, v2)
    a1 = ang(v1, v2, v0)
    a2 = ang(v2, v0, v1)
    u = sub3(v1, v0)
    w = sub3(v2, v0)
    cx = u[1] * w[2] - u[2] * w[1]
    cy = u[2] * w[0] - u[0] * w[2]
    cz = u[0] * w[1] - u[1] * w[0]
    cn = jnp.sqrt((cx * cx + cy * cy) + cz * cz)
    area = 0.5 * cn
    nx = cx / (cn + eps)
    ny = cy / (cn + eps)
    nz = cz / (cn + eps)

    def quant(x, high, low):
        t = (x - low) / (high - low)
        return jnp.clip(jnp.floor(t * 128.0), 0.0, 127.0).astype(jnp.int32) + 1

    mask = fm > 0.5
    import numpy as _np
    rows = []
    for i in range(9):
        rows.append(jnp.where(mask, quant(row(i), 1.0, 0.0), 0))
    for a in (a0, a1, a2):
        rows.append(jnp.where(mask, quant(a, _np.pi, 0.0), 0) + 129)
    for nrm in (nx, ny, nz):
        rows.append(jnp.where(mask, quant(nrm, 1.0, -1.0), 0) + 258)
    rows.append(jnp.where(mask, quant(area, 0.5, 0.0), 0) + 387)
    out_ref[...] = jnp.concatenate(rows, axis=0)


def _proj_body(feats_ref, wp_ref, wsp_ref, wl_ref, wr_ref, aux_ref,
               q_ref, r_ref):
    fb = feats_ref[...].astype(jnp.bfloat16)
    x = jnp.dot(fb, wp_ref[...], preferred_element_type=jnp.float32)
    x = x + aux_ref[0:1, :]
    h = jax.nn.gelu(x)
    hb = h.astype(jnp.bfloat16)
    p = jnp.maximum(
        jnp.dot(hb, wsp_ref[...], preferred_element_type=jnp.float32)
        + aux_ref[1:2, :], 0.0)
    q_ref[...] = jnp.dot(p.astype(jnp.bfloat16), wl_ref[...],
                         preferred_element_type=jnp.float32)
    r_ref[...] = jnp.dot(hb, wr_ref[...],
                         preferred_element_type=jnp.float32).astype(
                             jnp.bfloat16)


def _final_body(agg_ref, r_ref, cnt_ref, fm_ref, aux_ref, out_ref):
    x = agg_ref[...] / jnp.maximum(cnt_ref[...], 1.0)
    x = x + aux_ref[0:1, :] + r_ref[...].astype(jnp.float32)
    nrm = jnp.sqrt(jnp.sum(x * x, axis=1, keepdims=True))
    x = x / jnp.maximum(nrm, 1e-12)
    x = jax.nn.gelu(x)
    mu = jnp.mean(x, axis=1, keepdims=True)
    var = jnp.mean((x - mu) ** 2, axis=1, keepdims=True)
    x = (x - mu) / jnp.sqrt(var + 1e-5) * aux_ref[1:2, :] + aux_ref[2:3, :]
    out_ref[...] = x * fm_ref[...]


def kernel(vertices, faces, edges, face_masks, edge_masks, embed_vertex,
           embed_angle, embed_norm, embed_area, proj_W, proj_b, sage_proj_W,
           sage_proj_b, sage_Wl, sage_bl, sage_Wr, ln_gamma, ln_beta):
    v = vertices[0]
    vx, vy, vz = v[:, 0], v[:, 1], v[:, 2]
    fT = jnp.pad(jnp.transpose(faces[0]).astype(jnp.int32),
                 ((0, 0), (0, NPF - NF))).reshape(-1)
    fc9 = _vertex_gather(vx, vy, vz, fT).reshape(9, NPF)

    fmp = jnp.pad(face_masks[0].astype(jnp.float32), (0, NPF - NF))
    fc10 = jnp.concatenate([fc9, fmp[None, :]], axis=0)

    gidx = pl.pallas_call(
        _geom_body,
        grid=(NBLK,),
        in_specs=[pl.BlockSpec((10, BLK), lambda i: (0, i))],
        out_specs=pl.BlockSpec((16, BLK), lambda i: (0, i)),
        out_shape=jax.ShapeDtypeStruct((16, NPF), jnp.int32),
    )(fc10)

    gflat = jnp.transpose(gidx).reshape(-1)
    tbl = jnp.concatenate(
        [embed_vertex, embed_angle, embed_norm, embed_area], axis=0)
    feats = _embed_gather(jnp.tile(tbl, (16, 1)),
                          gflat).reshape(NPF, 16 * EDIM)

    aux_d = jnp.zeros((8, H), jnp.float32).at[0].set(proj_b).at[1].set(
        sage_proj_b)
    q, r = pl.pallas_call(
        _proj_body,
        grid=(NBLK,),
        in_specs=[
            pl.BlockSpec((BLK, 16 * EDIM), lambda i: (i, 0)),
            pl.BlockSpec((16 * EDIM, H), lambda i: (0, 0)),
            pl.BlockSpec((H, H), lambda i: (0, 0)),
            pl.BlockSpec((H, H), lambda i: (0, 0)),
            pl.BlockSpec((H, H), lambda i: (0, 0)),
            pl.BlockSpec((8, H), lambda i: (0, 0)),
        ],
        out_specs=[pl.BlockSpec((BLK, H), lambda i: (i, 0))] * 2,
        out_shape=[jax.ShapeDtypeStruct((NPF, H), jnp.float32),
                   jax.ShapeDtypeStruct((NPF, H), jnp.bfloat16)],
    )(feats, proj_W.astype(jnp.bfloat16), sage_proj_W.astype(jnp.bfloat16),
      sage_Wl.astype(jnp.bfloat16), sage_Wr.astype(jnp.bfloat16), aux_d)

    e = edges[0].astype(jnp.int32)
    src_p = jnp.pad(e[:, 0], (0, NE_P - NE))
    dst_eff = jnp.where(edge_masks[0], e[:, 1], 61440)
    dst_p = jnp.pad(dst_eff, (0, NE_P - NE), constant_values=61440)
    aggv, cnt2, _, _ = _edge_agg(q, src_p, dst_p)
    cntv = cnt2[:, 0]

    aux_f = jnp.zeros((8, H), jnp.float32).at[0].set(sage_bl).at[1].set(
        ln_gamma).at[2].set(ln_beta)
    out = pl.pallas_call(
        _final_body,
        grid=(NBLK,),
        in_specs=[
            pl.BlockSpec((BLK, H), lambda i: (i, 0)),
            pl.BlockSpec((BLK, H), lambda i: (i, 0)),
            pl.BlockSpec((BLK, 1), lambda i: (i, 0)),
            pl.BlockSpec((BLK, 1), lambda i: (i, 0)),
            pl.BlockSpec((8, H), lambda i: (0, 0)),
        ],
        out_specs=pl.BlockSpec((BLK, H), lambda i: (i, 0)),
        out_shape=jax.ShapeDtypeStruct((NPF, H), jnp.float32),
    )(aggv, r, cntv.reshape(NPF, 1), fmp.reshape(NPF, 1), aux_f)

    return out[:NF].reshape(1, NF, H)
